# baseline (speedup 1.0000x reference)
import functools

import jax
import jax.numpy as jnp
from jax import lax
from jax.experimental import pallas as pl
from jax.experimental.pallas import tpu as pltpu
from jax.experimental.pallas import tpu_sc as plsc

N = 10000
NPAD = 10112
NTILES = 16
RPT = NPAD // NTILES
E = 320000
CH = 128
CPT_FULL = 160
CPT_HALF = 80
IBLK = 40
ECHUNKS = 2 * NTILES * CPT_HALF
EPAD = ECHUNKS * CH
D_IN = 128
D1 = 256
D2 = 128
H1 = D1 // 2
EPS = 1e-5

_f32 = jnp.float32


def _sc_mesh():
    return plsc.VectorSubcoreMesh(
        core_axis_name="c", subcore_axis_name="s", num_cores=2, num_subcores=16
    )


@functools.partial(
    pl.kernel,
    out_type=(
        jax.ShapeDtypeStruct((NPAD, 128), _f32),
        jax.ShapeDtypeStruct((NPAD, 128), _f32),
    ),
    mesh=_sc_mesh(),
    scratch_types=(
        pltpu.VMEM((CPT_HALF, CH), jnp.int32),
        pltpu.VMEM((CH, 128), _f32),
        pltpu.VMEM_SHARED((NPAD, 128), _f32),
        pltpu.SemaphoreType.DMA,
    ),
)
def _deg_kernel(dst2d, ones_h, z128, p0_out, p1_out, didx, ones_v, dacc, dsem):
    c = lax.axis_index("c")
    s = lax.axis_index("s")
    row0 = pl.multiple_of(s * RPT, 8)
    pltpu.sync_copy(z128.at[pl.ds(row0, RPT)], dacc.at[pl.ds(row0, RPT)])
    pltpu.sync_copy(ones_h, ones_v)
    chunk0 = pl.multiple_of(c * (ECHUNKS // 2) + s * CPT_HALF, 8)
    pltpu.sync_copy(dst2d.at[pl.ds(chunk0, CPT_HALF)], didx)
    plsc.subcore_barrier()

    def batch(b, carry):
        def fire(i, carry2):
            pltpu.async_copy(ones_v, dacc.at[didx.at[b * 8 + i]], dsem, add=True)
            return carry2

        lax.fori_loop(0, 8, fire, 0)

        def drain(i, carry2):
            pltpu.make_async_copy(ones_v, dacc.at[didx.at[0]], dsem).wait()
            return carry2

        lax.fori_loop(0, 8, drain, 0)
        return carry

    lax.fori_loop(0, CPT_HALF // 8, batch, 0)
    plsc.subcore_barrier()

    @pl.when(c == 0)
    def _():
        pltpu.sync_copy(dacc.at[pl.ds(row0, RPT)], p0_out.at[pl.ds(row0, RPT)])

    @pl.when(c == 1)
    def _():
        pltpu.sync_copy(dacc.at[pl.ds(row0, RPT)], p1_out.at[pl.ds(row0, RPT)])


def _agg_pipeline(tbl, acc, src2d, dst2d, chunk0, nchunks,
                  sidx, didx, rows0, rows1, sem0, sem1, ssem0, ssem1):
    def block(b, carry):
        boff = pl.multiple_of(chunk0 + b * IBLK, 8)
        pltpu.sync_copy(src2d.at[pl.ds(boff, IBLK)], sidx)
        pltpu.sync_copy(dst2d.at[pl.ds(boff, IBLK)], didx)
        pltpu.async_copy(tbl.at[sidx.at[0]], rows0, sem0)
        pltpu.async_copy(tbl.at[sidx.at[1]], rows1, sem1)

        def pair(j, carry2):
            k0 = 2 * j
            k1 = k0 + 1
            pltpu.make_async_copy(tbl.at[sidx.at[k0]], rows0, sem0).wait()
            pltpu.async_copy(rows0, acc.at[didx.at[k0]], ssem0, add=True)
            pltpu.make_async_copy(tbl.at[sidx.at[k1]], rows1, sem1).wait()
            pltpu.async_copy(rows1, acc.at[didx.at[k1]], ssem1, add=True)

            @pl.when(k0 + 2 < IBLK)
            def _():
                pltpu.make_async_copy(rows0, acc.at[didx.at[k0]], ssem0).wait()
                pltpu.async_copy(tbl.at[sidx.at[k0 + 2]], rows0, sem0)
                pltpu.make_async_copy(rows1, acc.at[didx.at[k1]], ssem1).wait()
                pltpu.async_copy(tbl.at[sidx.at[k1 + 2]], rows1, sem1)

            return carry2

        lax.fori_loop(0, IBLK // 2, pair, 0)
        pltpu.make_async_copy(rows0, acc.at[didx.at[0]], ssem0).wait()
        pltpu.make_async_copy(rows1, acc.at[didx.at[0]], ssem1).wait()
        return carry

    lax.fori_loop(0, nchunks // IBLK, block, 0)


@functools.partial(
    pl.kernel,
    out_type=(
        jax.ShapeDtypeStruct((NPAD, H1), _f32),
        jax.ShapeDtypeStruct((NPAD, H1), _f32),
    ),
    mesh=_sc_mesh(),
    scratch_types=(
        pltpu.VMEM((IBLK, CH), jnp.int32),
        pltpu.VMEM((IBLK, CH), jnp.int32),
        pltpu.VMEM((CH, H1), _f32),
        pltpu.VMEM((CH, H1), _f32),
        pltpu.VMEM_SHARED((NPAD, H1), _f32),
        pltpu.SemaphoreType.DMA,
        pltpu.SemaphoreType.DMA,
        pltpu.SemaphoreType.DMA,
        pltpu.SemaphoreType.DMA,
    ),
)
def _agg_l1(src2d, dst2d, tlo, thi, zw, out_lo, out_hi,
            sidx, didx, rows0, rows1, acc, sem0, sem1, ssem0, ssem1):
    c = lax.axis_index("c")
    s = lax.axis_index("s")
    row0 = pl.multiple_of(s * RPT, 8)
    pltpu.sync_copy(zw.at[pl.ds(row0, RPT)], acc.at[pl.ds(row0, RPT)])
    chunk0 = pl.multiple_of(s * CPT_FULL, 8)
    plsc.subcore_barrier()

    @pl.when(c == 0)
    def _():
        _agg_pipeline(tlo, acc, src2d, dst2d, chunk0, CPT_FULL,
                      sidx, didx, rows0, rows1, sem0, sem1, ssem0, ssem1)

    @pl.when(c == 1)
    def _():
        _agg_pipeline(thi, acc, src2d, dst2d, chunk0, CPT_FULL,
                      sidx, didx, rows0, rows1, sem0, sem1, ssem0, ssem1)

    plsc.subcore_barrier()

    @pl.when(c == 0)
    def _():
        pltpu.sync_copy(acc.at[pl.ds(row0, RPT)], out_lo.at[pl.ds(row0, RPT)])

    @pl.when(c == 1)
    def _():
        pltpu.sync_copy(acc.at[pl.ds(row0, RPT)], out_hi.at[pl.ds(row0, RPT)])


@functools.partial(
    pl.kernel,
    out_type=(
        jax.ShapeDtypeStruct((NPAD, D2), _f32),
        jax.ShapeDtypeStruct((NPAD, D2), _f32),
    ),
    mesh=_sc_mesh(),
    scratch_types=(
        pltpu.VMEM((IBLK, CH), jnp.int32),
        pltpu.VMEM((IBLK, CH), jnp.int32),
        pltpu.VMEM((CH, D2), _f32),
        pltpu.VMEM((CH, D2), _f32),
        pltpu.VMEM_SHARED((NPAD, D2), _f32),
        pltpu.SemaphoreType.DMA,
        pltpu.SemaphoreType.DMA,
        pltpu.SemaphoreType.DMA,
        pltpu.SemaphoreType.DMA,
    ),
)
def _agg_l2(src2d, dst2d, tbl, zw, part0_out, part1_out,
            sidx, didx, rows0, rows1, acc, sem0, sem1, ssem0, ssem1):
    c = lax.axis_index("c")
    s = lax.axis_index("s")
    row0 = pl.multiple_of(s * RPT, 8)
    pltpu.sync_copy(zw.at[pl.ds(row0, RPT)], acc.at[pl.ds(row0, RPT)])
    chunk0 = pl.multiple_of(c * (ECHUNKS // 2) + s * CPT_HALF, 8)
    plsc.subcore_barrier()

    _agg_pipeline(tbl, acc, src2d, dst2d, chunk0, CPT_HALF,
                  sidx, didx, rows0, rows1, sem0, sem1, ssem0, ssem1)

    plsc.subcore_barrier()

    @pl.when(c == 0)
    def _():
        pltpu.sync_copy(acc.at[pl.ds(row0, RPT)], part0_out.at[pl.ds(row0, RPT)])

    @pl.when(c == 1)
    def _():
        pltpu.sync_copy(acc.at[pl.ds(row0, RPT)], part1_out.at[pl.ds(row0, RPT)])


def _mm1_body(x_ref, w1_ref, h_ref):
    h_ref[0:N, :] = jnp.dot(x_ref[...], w1_ref[...], preferred_element_type=_f32)
    h_ref[N:NPAD, :] = jnp.zeros((NPAD - N, D1), _f32)


_mm1 = pl.pallas_call(
    _mm1_body,
    out_shape=jax.ShapeDtypeStruct((NPAD, D1), _f32),
)


def _scale1_body(h_ref, p0_ref, p1_ref, hdlo_ref, hdhi_ref, dinv_ref):
    deg = p0_ref[:, 0:1] + p1_ref[:, 0:1] + 1.0
    dinv = lax.rsqrt(deg)
    dinv_ref[...] = dinv
    hd = h_ref[...] * dinv
    hdlo_ref[...] = hd[:, 0:H1]
    hdhi_ref[...] = hd[:, H1:D1]


_scale1 = pl.pallas_call(
    _scale1_body,
    out_shape=(
        jax.ShapeDtypeStruct((NPAD, H1), _f32),
        jax.ShapeDtypeStruct((NPAD, H1), _f32),
        jax.ShapeDtypeStruct((NPAD, 1), _f32),
    ),
)


def _norm1_body(acclo, acchi, hdlo, hdhi, dinv_ref, b_ref, w_ref, bias_ref,
                ms_ref, w2_ref, hd2_ref):
    dinv = dinv_ref[...]
    zlo = dinv * (acclo[...] + hdlo[...])
    zhi = dinv * (acchi[...] + hdhi[...])
    z = jnp.concatenate([zlo, zhi], axis=1)[0:N] + b_ref[...]
    mean = jnp.mean(z, axis=0, keepdims=True)
    cent = z - mean * ms_ref[...]
    var = jnp.mean(cent * cent, axis=0, keepdims=True)
    y = w_ref[...] * cent * lax.rsqrt(var + EPS) + bias_ref[...]
    y = jnp.maximum(y, 0.0)
    h2 = jnp.dot(y, w2_ref[...], preferred_element_type=_f32)
    hd2_ref[0:N, :] = h2 * dinv[0:N]
    hd2_ref[N:NPAD, :] = jnp.zeros((NPAD - N, D2), _f32)


_norm1 = pl.pallas_call(
    _norm1_body,
    out_shape=jax.ShapeDtypeStruct((NPAD, D2), _f32),
)


def _norm2_body(part0, part1, hd2, dinv_ref, b_ref, w_ref, bias_ref, ms_ref,
                out_ref):
    dinv = dinv_ref[...]
    acc = part0[...] + part1[...] + hd2[...]
    z = (dinv * acc)[0:N] + b_ref[...]
    mean = jnp.mean(z, axis=0, keepdims=True)
    cent = z - mean * ms_ref[...]
    var = jnp.mean(cent * cent, axis=0, keepdims=True)
    y = w_ref[...] * cent * lax.rsqrt(var + EPS) + bias_ref[...]
    out_ref[...] = jnp.maximum(y, 0.0)


_norm2 = pl.pallas_call(
    _norm2_body,
    out_shape=jax.ShapeDtypeStruct((N, D2), _f32),
)


def kernel(x, edge_index, W1, b1, gn1_w, gn1_b, gn1_ms, W2, b2, gn2_w, gn2_b, gn2_ms):
    ei = edge_index.astype(jnp.int32)
    pad = N + jnp.arange(EPAD - E, dtype=jnp.int32) % (NPAD - N)
    src2d = jnp.concatenate([ei[0], pad]).reshape(ECHUNKS, CH)
    dst2d = jnp.concatenate([ei[1], pad]).reshape(ECHUNKS, CH)

    ones128 = jnp.ones((CH, 128), _f32)
    z128 = jnp.zeros((NPAD, 128), _f32)

    h1 = _mm1(x, W1)
    p0, p1 = _deg_kernel(dst2d, ones128, z128)
    hd1lo, hd1hi, dinv = _scale1(h1, p0, p1)
    acc1lo, acc1hi = _agg_l1(src2d, dst2d, hd1lo, hd1hi, z128)
    hd2 = _norm1(acc1lo, acc1hi, hd1lo, hd1hi, dinv,
                 b1.reshape(1, D1), gn1_w.reshape(1, D1),
                 gn1_b.reshape(1, D1), gn1_ms.reshape(1, D1), W2)
    part0, part1 = _agg_l2(src2d, dst2d, hd2, z128)
    out = _norm2(part0, part1, hd2, dinv,
                 b2.reshape(1, D2), gn2_w.reshape(1, D2),
                 gn2_b.reshape(1, D2), gn2_ms.reshape(1, D2))
    return out


# speedup vs baseline: 1.2324x; 1.2324x over previous
import functools

import jax
import jax.numpy as jnp
from jax import lax
from jax.experimental import pallas as pl
from jax.experimental.pallas import tpu as pltpu
from jax.experimental.pallas import tpu_sc as plsc

N = 10000
NPAD = 10112
NTILES = 16
RPT = NPAD // NTILES
E = 320000
CH = 128
CPT_FULL = 160
CPT_HALF = 80
IBLK = 40
ECHUNKS = 2 * NTILES * CPT_HALF
EPAD = ECHUNKS * CH
D_IN = 128
D1 = 256
D2 = 128
H1 = D1 // 2
EPS = 1e-5

_f32 = jnp.float32


def _sc_mesh():
    return plsc.VectorSubcoreMesh(
        core_axis_name="c", subcore_axis_name="s", num_cores=2, num_subcores=16
    )


@functools.partial(
    pl.kernel,
    out_type=(
        jax.ShapeDtypeStruct((NPAD, 128), _f32),
        jax.ShapeDtypeStruct((NPAD, 128), _f32),
    ),
    mesh=_sc_mesh(),
    scratch_types=(
        pltpu.VMEM((CPT_HALF, CH), jnp.int32),
        pltpu.VMEM((CH, 128), _f32),
        pltpu.VMEM_SHARED((NPAD, 128), _f32),
        pltpu.SemaphoreType.DMA,
    ),
)
def _deg_kernel(dst2d, ones_h, z128, p0_out, p1_out, didx, ones_v, dacc, dsem):
    c = lax.axis_index("c")
    s = lax.axis_index("s")
    row0 = pl.multiple_of(s * RPT, 8)
    pltpu.sync_copy(z128.at[pl.ds(row0, RPT)], dacc.at[pl.ds(row0, RPT)])
    pltpu.sync_copy(ones_h, ones_v)
    chunk0 = pl.multiple_of(c * (ECHUNKS // 2) + s * CPT_HALF, 8)
    pltpu.sync_copy(dst2d.at[pl.ds(chunk0, CPT_HALF)], didx)
    plsc.subcore_barrier()

    def chunk(k, carry):
        pltpu.sync_copy(ones_v, dacc.at[didx.at[k]], add=True)
        return carry

    lax.fori_loop(0, CPT_HALF, chunk, 0)
    plsc.subcore_barrier()

    @pl.when(c == 0)
    def _():
        pltpu.sync_copy(dacc.at[pl.ds(row0, RPT)], p0_out.at[pl.ds(row0, RPT)])

    @pl.when(c == 1)
    def _():
        pltpu.sync_copy(dacc.at[pl.ds(row0, RPT)], p1_out.at[pl.ds(row0, RPT)])


def _agg_pipeline(tbl, acc, src2d, dst2d, chunk0, nchunks,
                  sidx, didx, rows0, rows1, sem0, sem1, ssem0, ssem1):
    def block(b, carry):
        boff = pl.multiple_of(chunk0 + b * IBLK, 8)
        pltpu.sync_copy(src2d.at[pl.ds(boff, IBLK)], sidx)
        pltpu.sync_copy(dst2d.at[pl.ds(boff, IBLK)], didx)
        pltpu.async_copy(tbl.at[sidx.at[0]], rows0, sem0)

        def pair(j, carry2):
            k0 = 2 * j
            k1 = k0 + 1
            pltpu.async_copy(tbl.at[sidx.at[k1]], rows1, sem1)
            pltpu.make_async_copy(tbl.at[sidx.at[k0]], rows0, sem0).wait()
            pltpu.sync_copy(rows0, acc.at[didx.at[k0]], add=True)

            @pl.when(k0 + 2 < IBLK)
            def _():
                pltpu.async_copy(tbl.at[sidx.at[k0 + 2]], rows0, sem0)

            pltpu.make_async_copy(tbl.at[sidx.at[k1]], rows1, sem1).wait()
            pltpu.sync_copy(rows1, acc.at[didx.at[k1]], add=True)
            return carry2

        lax.fori_loop(0, IBLK // 2, pair, 0)
        return carry

    lax.fori_loop(0, nchunks // IBLK, block, 0)


@functools.partial(
    pl.kernel,
    out_type=(
        jax.ShapeDtypeStruct((NPAD, H1), _f32),
        jax.ShapeDtypeStruct((NPAD, H1), _f32),
    ),
    mesh=_sc_mesh(),
    scratch_types=(
        pltpu.VMEM((IBLK, CH), jnp.int32),
        pltpu.VMEM((IBLK, CH), jnp.int32),
        pltpu.VMEM((CH, H1), _f32),
        pltpu.VMEM((CH, H1), _f32),
        pltpu.VMEM_SHARED((NPAD, H1), _f32),
        pltpu.SemaphoreType.DMA,
        pltpu.SemaphoreType.DMA,
        pltpu.SemaphoreType.DMA,
        pltpu.SemaphoreType.DMA,
    ),
)
def _agg_l1(src2d, dst2d, tlo, thi, zw, out_lo, out_hi,
            sidx, didx, rows0, rows1, acc, sem0, sem1, ssem0, ssem1):
    c = lax.axis_index("c")
    s = lax.axis_index("s")
    row0 = pl.multiple_of(s * RPT, 8)
    pltpu.sync_copy(zw.at[pl.ds(row0, RPT)], acc.at[pl.ds(row0, RPT)])
    chunk0 = pl.multiple_of(s * CPT_FULL, 8)
    plsc.subcore_barrier()

    @pl.when(c == 0)
    def _():
        _agg_pipeline(tlo, acc, src2d, dst2d, chunk0, CPT_FULL,
                      sidx, didx, rows0, rows1, sem0, sem1, ssem0, ssem1)

    @pl.when(c == 1)
    def _():
        _agg_pipeline(thi, acc, src2d, dst2d, chunk0, CPT_FULL,
                      sidx, didx, rows0, rows1, sem0, sem1, ssem0, ssem1)

    plsc.subcore_barrier()

    @pl.when(c == 0)
    def _():
        pltpu.sync_copy(acc.at[pl.ds(row0, RPT)], out_lo.at[pl.ds(row0, RPT)])

    @pl.when(c == 1)
    def _():
        pltpu.sync_copy(acc.at[pl.ds(row0, RPT)], out_hi.at[pl.ds(row0, RPT)])


@functools.partial(
    pl.kernel,
    out_type=(
        jax.ShapeDtypeStruct((NPAD, D2), _f32),
        jax.ShapeDtypeStruct((NPAD, D2), _f32),
    ),
    mesh=_sc_mesh(),
    scratch_types=(
        pltpu.VMEM((IBLK, CH), jnp.int32),
        pltpu.VMEM((IBLK, CH), jnp.int32),
        pltpu.VMEM((CH, D2), _f32),
        pltpu.VMEM((CH, D2), _f32),
        pltpu.VMEM_SHARED((NPAD, D2), _f32),
        pltpu.SemaphoreType.DMA,
        pltpu.SemaphoreType.DMA,
        pltpu.SemaphoreType.DMA,
        pltpu.SemaphoreType.DMA,
    ),
)
def _agg_l2(src2d, dst2d, tbl, zw, part0_out, part1_out,
            sidx, didx, rows0, rows1, acc, sem0, sem1, ssem0, ssem1):
    c = lax.axis_index("c")
    s = lax.axis_index("s")
    row0 = pl.multiple_of(s * RPT, 8)
    pltpu.sync_copy(zw.at[pl.ds(row0, RPT)], acc.at[pl.ds(row0, RPT)])
    chunk0 = pl.multiple_of(c * (ECHUNKS // 2) + s * CPT_HALF, 8)
    plsc.subcore_barrier()

    _agg_pipeline(tbl, acc, src2d, dst2d, chunk0, CPT_HALF,
                  sidx, didx, rows0, rows1, sem0, sem1, ssem0, ssem1)

    plsc.subcore_barrier()

    @pl.when(c == 0)
    def _():
        pltpu.sync_copy(acc.at[pl.ds(row0, RPT)], part0_out.at[pl.ds(row0, RPT)])

    @pl.when(c == 1)
    def _():
        pltpu.sync_copy(acc.at[pl.ds(row0, RPT)], part1_out.at[pl.ds(row0, RPT)])


def _mm1_body(x_ref, w1_ref, h_ref):
    h_ref[0:N, :] = jnp.dot(x_ref[...], w1_ref[...], preferred_element_type=_f32)
    h_ref[N:NPAD, :] = jnp.zeros((NPAD - N, D1), _f32)


_mm1 = pl.pallas_call(
    _mm1_body,
    out_shape=jax.ShapeDtypeStruct((NPAD, D1), _f32),
)


def _scale1_body(h_ref, p0_ref, p1_ref, hdlo_ref, hdhi_ref, dinv_ref):
    deg = p0_ref[:, 0:1] + p1_ref[:, 0:1] + 1.0
    dinv = lax.rsqrt(deg)
    dinv_ref[...] = dinv
    hd = h_ref[...] * dinv
    hdlo_ref[...] = hd[:, 0:H1]
    hdhi_ref[...] = hd[:, H1:D1]


_scale1 = pl.pallas_call(
    _scale1_body,
    out_shape=(
        jax.ShapeDtypeStruct((NPAD, H1), _f32),
        jax.ShapeDtypeStruct((NPAD, H1), _f32),
        jax.ShapeDtypeStruct((NPAD, 1), _f32),
    ),
)


def _norm1_body(acclo, acchi, hdlo, hdhi, dinv_ref, b_ref, w_ref, bias_ref,
                ms_ref, w2_ref, hd2_ref):
    dinv = dinv_ref[...]
    zlo = dinv * (acclo[...] + hdlo[...])
    zhi = dinv * (acchi[...] + hdhi[...])
    z = jnp.concatenate([zlo, zhi], axis=1)[0:N] + b_ref[...]
    mean = jnp.mean(z, axis=0, keepdims=True)
    cent = z - mean * ms_ref[...]
    var = jnp.mean(cent * cent, axis=0, keepdims=True)
    y = w_ref[...] * cent * lax.rsqrt(var + EPS) + bias_ref[...]
    y = jnp.maximum(y, 0.0)
    h2 = jnp.dot(y, w2_ref[...], preferred_element_type=_f32)
    hd2_ref[0:N, :] = h2 * dinv[0:N]
    hd2_ref[N:NPAD, :] = jnp.zeros((NPAD - N, D2), _f32)


_norm1 = pl.pallas_call(
    _norm1_body,
    out_shape=jax.ShapeDtypeStruct((NPAD, D2), _f32),
)


def _norm2_body(part0, part1, hd2, dinv_ref, b_ref, w_ref, bias_ref, ms_ref,
                out_ref):
    dinv = dinv_ref[...]
    acc = part0[...] + part1[...] + hd2[...]
    z = (dinv * acc)[0:N] + b_ref[...]
    mean = jnp.mean(z, axis=0, keepdims=True)
    cent = z - mean * ms_ref[...]
    var = jnp.mean(cent * cent, axis=0, keepdims=True)
    y = w_ref[...] * cent * lax.rsqrt(var + EPS) + bias_ref[...]
    out_ref[...] = jnp.maximum(y, 0.0)


_norm2 = pl.pallas_call(
    _norm2_body,
    out_shape=jax.ShapeDtypeStruct((N, D2), _f32),
)


def kernel(x, edge_index, W1, b1, gn1_w, gn1_b, gn1_ms, W2, b2, gn2_w, gn2_b, gn2_ms):
    ei = edge_index.astype(jnp.int32)
    pad = N + jnp.arange(EPAD - E, dtype=jnp.int32) % (NPAD - N)
    src2d = jnp.concatenate([ei[0], pad]).reshape(ECHUNKS, CH)
    dst2d = jnp.concatenate([ei[1], pad]).reshape(ECHUNKS, CH)

    ones128 = jnp.ones((CH, 128), _f32)
    z128 = jnp.zeros((NPAD, 128), _f32)

    h1 = _mm1(x, W1)
    p0, p1 = _deg_kernel(dst2d, ones128, z128)
    hd1lo, hd1hi, dinv = _scale1(h1, p0, p1)
    acc1lo, acc1hi = _agg_l1(src2d, dst2d, hd1lo, hd1hi, z128)
    hd2 = _norm1(acc1lo, acc1hi, hd1lo, hd1hi, dinv,
                 b1.reshape(1, D1), gn1_w.reshape(1, D1),
                 gn1_b.reshape(1, D1), gn1_ms.reshape(1, D1), W2)
    part0, part1 = _agg_l2(src2d, dst2d, hd2, z128)
    out = _norm2(part0, part1, hd2, dinv,
                 b2.reshape(1, D2), gn2_w.reshape(1, D2),
                 gn2_b.reshape(1, D2), gn2_ms.reshape(1, D2))
    return out


# speedup vs baseline: 1.2447x; 1.0100x over previous
import functools

import jax
import jax.numpy as jnp
from jax import lax
from jax.experimental import pallas as pl
from jax.experimental.pallas import tpu as pltpu
from jax.experimental.pallas import tpu_sc as plsc

N = 10000
NPAD = 10112
NTILES = 16
RPT = NPAD // NTILES
E = 320000
CH = 128
CPT_FULL = 160
CPT_HALF = 80
IBLK = 40
ECHUNKS = 2 * NTILES * CPT_HALF
EPAD = ECHUNKS * CH
D_IN = 128
D1 = 256
D2 = 128
H1 = D1 // 2
DEGW = 128
EPS = 1e-5

_f32 = jnp.float32


def _sc_mesh():
    return plsc.VectorSubcoreMesh(
        core_axis_name="c", subcore_axis_name="s", num_cores=2, num_subcores=16
    )


@functools.partial(
    pl.kernel,
    out_type=(
        jax.ShapeDtypeStruct((NPAD, DEGW), _f32),
        jax.ShapeDtypeStruct((NPAD, DEGW), _f32),
    ),
    mesh=_sc_mesh(),
    scratch_types=(
        pltpu.VMEM((CPT_HALF, CH), jnp.int32),
        pltpu.VMEM((CH, 128), _f32),
        pltpu.VMEM_SHARED((NPAD, 128), _f32),
    ),
)
def _deg_kernel(dst2d, ones_h, z128, p0_out, p1_out, didx, ones_v, dacc):
    c = lax.axis_index("c")
    s = lax.axis_index("s")
    row0 = pl.multiple_of(s * RPT, 8)
    pltpu.sync_copy(z128.at[pl.ds(row0, RPT)], dacc.at[pl.ds(row0, RPT)])
    pltpu.sync_copy(ones_h, ones_v)
    chunk0 = pl.multiple_of(c * (ECHUNKS // 2) + s * CPT_HALF, 8)
    pltpu.sync_copy(dst2d.at[pl.ds(chunk0, CPT_HALF)], didx)
    plsc.subcore_barrier()

    def chunk(k, carry):
        pltpu.sync_copy(ones_v, dacc.at[didx.at[k]], add=True)
        return carry

    lax.fori_loop(0, CPT_HALF, chunk, 0)
    plsc.subcore_barrier()

    @pl.when(c == 0)
    def _():
        pltpu.sync_copy(dacc.at[pl.ds(row0, RPT)], p0_out.at[pl.ds(row0, RPT)])

    @pl.when(c == 1)
    def _():
        pltpu.sync_copy(dacc.at[pl.ds(row0, RPT)], p1_out.at[pl.ds(row0, RPT)])


def _agg_pipeline(tbl, acc, src2d, dst2d, chunk0, nchunks,
                  sidx, didx, rows0, rows1, sem0, sem1):
    def block(b, carry):
        boff = pl.multiple_of(chunk0 + b * IBLK, 8)
        pltpu.sync_copy(src2d.at[pl.ds(boff, IBLK)], sidx)
        pltpu.sync_copy(dst2d.at[pl.ds(boff, IBLK)], didx)
        pltpu.async_copy(tbl.at[sidx.at[0]], rows0, sem0)

        def pair(j, carry2):
            k0 = 2 * j
            k1 = k0 + 1
            pltpu.async_copy(tbl.at[sidx.at[k1]], rows1, sem1)
            pltpu.make_async_copy(tbl.at[sidx.at[k0]], rows0, sem0).wait()
            pltpu.sync_copy(rows0, acc.at[didx.at[k0]], add=True)

            @pl.when(k0 + 2 < IBLK)
            def _():
                pltpu.async_copy(tbl.at[sidx.at[k0 + 2]], rows0, sem0)

            pltpu.make_async_copy(tbl.at[sidx.at[k1]], rows1, sem1).wait()
            pltpu.sync_copy(rows1, acc.at[didx.at[k1]], add=True)
            return carry2

        lax.fori_loop(0, IBLK // 2, pair, 0)
        return carry

    lax.fori_loop(0, nchunks // IBLK, block, 0)


@functools.partial(
    pl.kernel,
    out_type=(
        jax.ShapeDtypeStruct((NPAD, H1), _f32),
        jax.ShapeDtypeStruct((NPAD, H1), _f32),
    ),
    mesh=_sc_mesh(),
    scratch_types=(
        pltpu.VMEM((IBLK, CH), jnp.int32),
        pltpu.VMEM((IBLK, CH), jnp.int32),
        pltpu.VMEM((CH, H1), _f32),
        pltpu.VMEM((CH, H1), _f32),
        pltpu.VMEM_SHARED((NPAD, H1), _f32),
        pltpu.SemaphoreType.DMA,
        pltpu.SemaphoreType.DMA,
    ),
)
def _agg_l1(src2d, dst2d, tlo, thi, zw, out_lo, out_hi,
            sidx, didx, rows0, rows1, acc, sem0, sem1):
    c = lax.axis_index("c")
    s = lax.axis_index("s")
    row0 = pl.multiple_of(s * RPT, 8)
    pltpu.sync_copy(zw.at[pl.ds(row0, RPT)], acc.at[pl.ds(row0, RPT)])
    chunk0 = pl.multiple_of(s * CPT_FULL, 8)
    plsc.subcore_barrier()

    @pl.when(c == 0)
    def _():
        _agg_pipeline(tlo, acc, src2d, dst2d, chunk0, CPT_FULL,
                      sidx, didx, rows0, rows1, sem0, sem1)

    @pl.when(c == 1)
    def _():
        _agg_pipeline(thi, acc, src2d, dst2d, chunk0, CPT_FULL,
                      sidx, didx, rows0, rows1, sem0, sem1)

    plsc.subcore_barrier()

    @pl.when(c == 0)
    def _():
        pltpu.sync_copy(acc.at[pl.ds(row0, RPT)], out_lo.at[pl.ds(row0, RPT)])

    @pl.when(c == 1)
    def _():
        pltpu.sync_copy(acc.at[pl.ds(row0, RPT)], out_hi.at[pl.ds(row0, RPT)])


@functools.partial(
    pl.kernel,
    out_type=(
        jax.ShapeDtypeStruct((NPAD, D2), _f32),
        jax.ShapeDtypeStruct((NPAD, D2), _f32),
    ),
    mesh=_sc_mesh(),
    scratch_types=(
        pltpu.VMEM((IBLK, CH), jnp.int32),
        pltpu.VMEM((IBLK, CH), jnp.int32),
        pltpu.VMEM((CH, D2), _f32),
        pltpu.VMEM((CH, D2), _f32),
        pltpu.VMEM_SHARED((NPAD, D2), _f32),
        pltpu.SemaphoreType.DMA,
        pltpu.SemaphoreType.DMA,
    ),
)
def _agg_l2(src2d, dst2d, tbl, zw, part0_out, part1_out,
            sidx, didx, rows0, rows1, acc, sem0, sem1):
    c = lax.axis_index("c")
    s = lax.axis_index("s")
    row0 = pl.multiple_of(s * RPT, 8)
    pltpu.sync_copy(zw.at[pl.ds(row0, RPT)], acc.at[pl.ds(row0, RPT)])
    chunk0 = pl.multiple_of(c * (ECHUNKS // 2) + s * CPT_HALF, 8)
    plsc.subcore_barrier()

    _agg_pipeline(tbl, acc, src2d, dst2d, chunk0, CPT_HALF,
                  sidx, didx, rows0, rows1, sem0, sem1)

    plsc.subcore_barrier()

    @pl.when(c == 0)
    def _():
        pltpu.sync_copy(acc.at[pl.ds(row0, RPT)], part0_out.at[pl.ds(row0, RPT)])

    @pl.when(c == 1)
    def _():
        pltpu.sync_copy(acc.at[pl.ds(row0, RPT)], part1_out.at[pl.ds(row0, RPT)])


def _tc1_body(x_ref, w1_ref, p0_ref, p1_ref, hdlo_ref, hdhi_ref, dinv_ref):
    deg = p0_ref[:, 0:1] + p1_ref[:, 0:1] + 1.0
    dinv = lax.rsqrt(deg)
    dinv_ref[...] = dinv
    h = jnp.dot(x_ref[...], w1_ref[...], preferred_element_type=_f32)
    hd = h * dinv[0:N]
    zpad = jnp.zeros((NPAD - N, H1), _f32)
    hdlo_ref[0:N, :] = hd[:, 0:H1]
    hdlo_ref[N:NPAD, :] = zpad
    hdhi_ref[0:N, :] = hd[:, H1:D1]
    hdhi_ref[N:NPAD, :] = zpad


_tc1 = pl.pallas_call(
    _tc1_body,
    out_shape=(
        jax.ShapeDtypeStruct((NPAD, H1), _f32),
        jax.ShapeDtypeStruct((NPAD, H1), _f32),
        jax.ShapeDtypeStruct((NPAD, 1), _f32),
    ),
)


def _norm1_body(acclo, acchi, hdlo, hdhi, dinv_ref, b_ref, w_ref, bias_ref,
                ms_ref, w2_ref, hd2_ref):
    dinv = dinv_ref[...]
    zlo = dinv * (acclo[...] + hdlo[...])
    zhi = dinv * (acchi[...] + hdhi[...])
    z = jnp.concatenate([zlo, zhi], axis=1)[0:N] + b_ref[...]
    mean = jnp.mean(z, axis=0, keepdims=True)
    cent = z - mean * ms_ref[...]
    var = jnp.mean(cent * cent, axis=0, keepdims=True)
    y = w_ref[...] * cent * lax.rsqrt(var + EPS) + bias_ref[...]
    y = jnp.maximum(y, 0.0)
    h2 = jnp.dot(y, w2_ref[...], preferred_element_type=_f32)
    hd2_ref[0:N, :] = h2 * dinv[0:N]
    hd2_ref[N:NPAD, :] = jnp.zeros((NPAD - N, D2), _f32)


_norm1 = pl.pallas_call(
    _norm1_body,
    out_shape=jax.ShapeDtypeStruct((NPAD, D2), _f32),
)


def _norm2_body(part0, part1, hd2, dinv_ref, b_ref, w_ref, bias_ref, ms_ref,
                out_ref):
    dinv = dinv_ref[...]
    acc = part0[...] + part1[...] + hd2[...]
    z = (dinv * acc)[0:N] + b_ref[...]
    mean = jnp.mean(z, axis=0, keepdims=True)
    cent = z - mean * ms_ref[...]
    var = jnp.mean(cent * cent, axis=0, keepdims=True)
    y = w_ref[...] * cent * lax.rsqrt(var + EPS) + bias_ref[...]
    out_ref[...] = jnp.maximum(y, 0.0)


_norm2 = pl.pallas_call(
    _norm2_body,
    out_shape=jax.ShapeDtypeStruct((N, D2), _f32),
)


def kernel(x, edge_index, W1, b1, gn1_w, gn1_b, gn1_ms, W2, b2, gn2_w, gn2_b, gn2_ms):
    ei = edge_index.astype(jnp.int32)
    pad = N + jnp.arange(EPAD - E, dtype=jnp.int32) % (NPAD - N)
    src2d = jnp.concatenate([ei[0], pad]).reshape(ECHUNKS, CH)
    dst2d = jnp.concatenate([ei[1], pad]).reshape(ECHUNKS, CH)

    ones128 = jnp.ones((CH, 128), _f32)
    z128 = jnp.zeros((NPAD, 128), _f32)

    p0, p1 = _deg_kernel(dst2d, ones128, z128)
    hd1lo, hd1hi, dinv = _tc1(x, W1, p0, p1)
    acc1lo, acc1hi = _agg_l1(src2d, dst2d, hd1lo, hd1hi, z128)
    hd2 = _norm1(acc1lo, acc1hi, hd1lo, hd1hi, dinv,
                 b1.reshape(1, D1), gn1_w.reshape(1, D1),
                 gn1_b.reshape(1, D1), gn1_ms.reshape(1, D1), W2)
    part0, part1 = _agg_l2(src2d, dst2d, hd2, z128)
    out = _norm2(part0, part1, hd2, dinv,
                 b2.reshape(1, D2), gn2_w.reshape(1, D2),
                 gn2_b.reshape(1, D2), gn2_ms.reshape(1, D2))
    return out
